# baseline (speedup 1.0000x reference)
import functools

import jax
import jax.numpy as jnp
import numpy as np
from jax import lax
from jax.experimental import pallas as pl
from jax.experimental.pallas import tpu as pltpu
from jax.experimental.pallas import tpu_sc as plsc

NB_CODE = 512
CODE_DIM = 512
WIDTH = 512
DEPTH = 3
DOWN_T = 3
DRATE = 3
INPUT_DIM = 251
N_TOK = 2048

PAD = 16
NROW = N_TOK + 2 * PAD

_NC = 2
_NS = 16
_NW = _NC * _NS
_B_PER_W = N_TOK // _NW

_DST_X = 0
_DST_T = 1
_DST_XADD = 2
_DST_OUT = 3


def _sc_gather(codebook, idx):
    mesh = plsc.VectorSubcoreMesh(core_axis_name="c", subcore_axis_name="s")

    @functools.partial(
        pl.kernel,
        out_type=jax.ShapeDtypeStruct((N_TOK, CODE_DIM), jnp.float32),
        mesh=mesh,
        scratch_types=[
            pltpu.VMEM((_B_PER_W,), jnp.int32),
            pltpu.VMEM((_B_PER_W, CODE_DIM), jnp.float32),
            pltpu.SemaphoreType.DMA,
        ],
    )
    def gather_kernel(table_hbm, idx_hbm, out_hbm, idx_v, rows_v, sem):
        wid = lax.axis_index("s") * _NC + lax.axis_index("c")
        base = wid * _B_PER_W
        pltpu.sync_copy(idx_hbm.at[pl.ds(base, _B_PER_W)], idx_v)
        pltpu.async_copy(table_hbm.at[idx_v], rows_v, sem).wait()
        pltpu.sync_copy(rows_v, out_hbm.at[pl.ds(base, _B_PER_W)])

    return gather_kernel(codebook, idx)


def _layer_schedule():
    layers = [(3, 1, 0, 0, _DST_X, 1)]
    dils = [DRATE ** d for d in range(DEPTH)][::-1]
    for _ in range(DOWN_T):
        for dil in dils:
            layers.append((3, dil, 0, 1, _DST_T, 0))
            layers.append((1, 1, 1, 1, _DST_XADD, 0))
        layers.append((3, 1, 0, 0, _DST_X, 0))
    layers.append((3, 1, 0, 0, _DST_X, 1))
    layers.append((3, 1, 0, 0, _DST_OUT, 0))
    return layers


def _build_ctrl():
    rows = []
    for li, (k, dil, src, pre, dst, post) in enumerate(_layer_schedule()):
        shifts = [-dil, 0, dil] if k == 3 else [0]
        for j, s in enumerate(shifts):
            rows.append([s, src, pre, int(j == 0), int(j == len(shifts) - 1),
                         dst, post, li])
    return np.asarray(rows, dtype=np.int32)


_CTRL = _build_ctrl()
_SHIFTS = tuple(sorted(set(_CTRL[:, 0].tolist())))
_N_TAPS = _CTRL.shape[0]
_N_LAYERS = len(_layer_schedule())


def _param_list(params):
    out = [(params['conv_in']['w'], params['conv_in']['b'])]
    for blk in params['blocks']:
        for rb in blk['res']:
            out.append((rb['c1']['w'], rb['c1']['b']))
            out.append((rb['c2']['w'], rb['c2']['b']))
        out.append((blk['conv']['w'], blk['conv']['b']))
    out.append((params['conv_mid']['w'], params['conv_mid']['b']))
    out.append((params['conv_out']['w'], params['conv_out']['b']))
    return out


def _pack_weights(params):
    taps = []
    biases = []
    for (w, b) in _param_list(params):
        o = w.shape[0]
        if o < WIDTH:
            w = jnp.pad(w, ((0, WIDTH - o), (0, 0), (0, 0)))
            b = jnp.pad(b, (0, WIDTH - o))
        for j in range(w.shape[-1]):
            taps.append(w[:, :, j].T)
        biases.append(b)
    return jnp.stack(taps), jnp.stack(biases)


def _tap_body(ctrl_ref, g_ref, w_ref, b_ref, out_ref, x_ref, t_ref, acc_ref):
    i = pl.program_id(0)
    shift = ctrl_ref[i, 0]
    src = ctrl_ref[i, 1]
    pre = ctrl_ref[i, 2]
    first = ctrl_ref[i, 3]
    last = ctrl_ref[i, 4]
    dst = ctrl_ref[i, 5]
    post = ctrl_ref[i, 6]
    layer = ctrl_ref[i, 7]

    @pl.when(i == 0)
    def _init():
        x_ref[...] = jnp.zeros((NROW, WIDTH), jnp.float32)
        t_ref[...] = jnp.zeros((NROW, WIDTH), jnp.float32)
        x_ref[PAD:PAD + N_TOK, :] = g_ref[...]

    w = w_ref[0]

    def do_tap(src_ref, s):
        v = src_ref[PAD + s:PAD + s + N_TOK, :]
        v = jnp.where(pre == 1, jnp.maximum(v, 0.0), v)
        contrib = jnp.dot(v, w, preferred_element_type=jnp.float32)

        @pl.when(first == 1)
        def _():
            acc_ref[...] = contrib

        @pl.when(first == 0)
        def _():
            acc_ref[...] += contrib

    for s in _SHIFTS:
        @pl.when(shift == s)
        def _(s=s):
            if s == 0:
                @pl.when(src == 0)
                def _():
                    do_tap(x_ref, s)

                @pl.when(src == 1)
                def _():
                    do_tap(t_ref, s)
            else:
                do_tap(x_ref, s)

    @pl.when(last == 1)
    def _finish():
        bias = b_ref[pl.ds(layer, 1), :]
        val = acc_ref[...] + bias
        val = jnp.where(post == 1, jnp.maximum(val, 0.0), val)

        @pl.when(dst == _DST_X)
        def _():
            x_ref[PAD:PAD + N_TOK, :] = val

        @pl.when(dst == _DST_T)
        def _():
            t_ref[PAD:PAD + N_TOK, :] = val

        @pl.when(dst == _DST_XADD)
        def _():
            x_ref[PAD:PAD + N_TOK, :] += val

        @pl.when(dst == _DST_OUT)
        def _():
            out_ref[...] = val


def _tc_decode(g, big_w, big_b):
    ctrl = jnp.asarray(_CTRL)
    return pl.pallas_call(
        _tap_body,
        grid=(_N_TAPS,),
        in_specs=[
            pl.BlockSpec(memory_space=pltpu.SMEM),
            pl.BlockSpec((N_TOK, WIDTH), lambda i: (0, 0)),
            pl.BlockSpec((1, WIDTH, WIDTH), lambda i: (i, 0, 0)),
            pl.BlockSpec((_N_LAYERS, WIDTH), lambda i: (0, 0)),
        ],
        out_specs=pl.BlockSpec((N_TOK, WIDTH), lambda i: (0, 0)),
        out_shape=jax.ShapeDtypeStruct((N_TOK, WIDTH), jnp.float32),
        scratch_shapes=[
            pltpu.VMEM((NROW, WIDTH), jnp.float32),
            pltpu.VMEM((NROW, WIDTH), jnp.float32),
            pltpu.VMEM((N_TOK, WIDTH), jnp.float32),
        ],
        compiler_params=pltpu.CompilerParams(
            dimension_semantics=("arbitrary",),
        ),
    )(ctrl, g, big_w, big_b)


def kernel(x, codebook, params):
    idx = x.astype(jnp.int32)
    g = _sc_gather(codebook, idx)
    big_w, big_b = _pack_weights(params)
    out = _tc_decode(g, big_w, big_b)
    return out[:, :INPUT_DIM].reshape(1, N_TOK, INPUT_DIM)


# speedup vs baseline: 1.2830x; 1.2830x over previous
import functools

import jax
import jax.numpy as jnp
import numpy as np
from jax import lax
from jax.experimental import pallas as pl
from jax.experimental.pallas import tpu as pltpu
from jax.experimental.pallas import tpu_sc as plsc

NB_CODE = 512
CODE_DIM = 512
WIDTH = 512
DEPTH = 3
DOWN_T = 3
DRATE = 3
INPUT_DIM = 251
N_TOK = 2048

PAD = 16
NROW = N_TOK + 2 * PAD
K3 = 3 * WIDTH

_NC = 2
_NS = 16
_NW = _NC * _NS
_B_PER_W = N_TOK // _NW

_DILS = tuple(DRATE ** d for d in range(DEPTH))[::-1]

_K_PLAIN_X = 0
_K_PLAIN_OUT = 1
_K_RES1 = 2
_K_RES2 = 3


def _sc_gather(codebook, idx):
    mesh = plsc.VectorSubcoreMesh(core_axis_name="c", subcore_axis_name="s")

    @functools.partial(
        pl.kernel,
        out_type=jax.ShapeDtypeStruct((N_TOK, CODE_DIM), jnp.float32),
        mesh=mesh,
        scratch_types=[
            pltpu.VMEM((_B_PER_W,), jnp.int32),
            pltpu.VMEM((_B_PER_W, CODE_DIM), jnp.float32),
            pltpu.SemaphoreType.DMA,
        ],
    )
    def gather_kernel(table_hbm, idx_hbm, out_hbm, idx_v, rows_v, sem):
        wid = lax.axis_index("s") * _NC + lax.axis_index("c")
        base = wid * _B_PER_W
        pltpu.sync_copy(idx_hbm.at[pl.ds(base, _B_PER_W)], idx_v)
        pltpu.async_copy(table_hbm.at[idx_v], rows_v, sem).wait()
        pltpu.sync_copy(rows_v, out_hbm.at[pl.ds(base, _B_PER_W)])

    return gather_kernel(codebook, idx)


def _layer_schedule():
    layers = [(_K_PLAIN_X, 1, 1)]
    for _ in range(DOWN_T):
        for dil in _DILS:
            layers.append((_K_RES1, dil, 0))
            layers.append((_K_RES2, 1, 0))
        layers.append((_K_PLAIN_X, 1, 0))
    layers.append((_K_PLAIN_X, 1, 1))
    layers.append((_K_PLAIN_OUT, 1, 0))
    return layers


_LAYERS = _layer_schedule()
_N_LAYERS = len(_LAYERS)
_CTRL = np.asarray([[k, d, p] for (k, d, p) in _LAYERS], dtype=np.int32)


def _param_list(params):
    out = [(params['conv_in']['w'], params['conv_in']['b'])]
    for blk in params['blocks']:
        for rb in blk['res']:
            out.append((rb['c1']['w'], rb['c1']['b']))
            out.append((rb['c2']['w'], rb['c2']['b']))
        out.append((blk['conv']['w'], blk['conv']['b']))
    out.append((params['conv_mid']['w'], params['conv_mid']['b']))
    out.append((params['conv_out']['w'], params['conv_out']['b']))
    return out


def _pack_weights(params):
    ws = []
    biases = []
    for (w, b) in _param_list(params):
        o = w.shape[0]
        if o < WIDTH:
            w = jnp.pad(w, ((0, WIDTH - o), (0, 0), (0, 0)))
            b = jnp.pad(b, (0, WIDTH - o))
        k = w.shape[-1]
        if k == 3:
            wl = jnp.concatenate([w[:, :, 0].T, w[:, :, 1].T, w[:, :, 2].T], axis=0)
        else:
            z = jnp.zeros((WIDTH, WIDTH), jnp.float32)
            wl = jnp.concatenate([z, w[:, :, 0].T, z], axis=0)
        ws.append(wl)
        biases.append(b)
    return jnp.stack(ws), jnp.stack(biases)


def _layer_body(ctrl_ref, g_ref, w_ref, b_ref, out_ref, x_ref, t_ref, cat_ref):
    i = pl.program_id(0)
    kind = ctrl_ref[i, 0]
    dil = ctrl_ref[i, 1]
    post = ctrl_ref[i, 2]

    @pl.when(i == 0)
    def _init():
        x_ref[...] = jnp.zeros((NROW, WIDTH), jnp.float32)
        t_ref[...] = jnp.zeros((NROW, WIDTH), jnp.float32)
        x_ref[PAD:PAD + N_TOK, :] = g_ref[...]

    bias = b_ref[pl.ds(i, 1), :]

    def finish_plain(val):
        val = jnp.where(post == 1, jnp.maximum(val, 0.0), val)

        @pl.when(kind == _K_PLAIN_X)
        def _():
            x_ref[PAD:PAD + N_TOK, :] = val

        @pl.when(kind == _K_PLAIN_OUT)
        def _():
            out_ref[...] = val

    def build_cat(src_ref, d, with_relu):
        for j, s in enumerate((-d, 0, d)):
            v = src_ref[PAD + s:PAD + s + N_TOK, :]
            if with_relu:
                v = jnp.maximum(v, 0.0)
            cat_ref[:, j * WIDTH:(j + 1) * WIDTH] = v

    @pl.when(kind <= _K_PLAIN_OUT)
    def _plain():
        build_cat(x_ref, 1, False)
        val = jnp.dot(cat_ref[...], w_ref[0],
                      preferred_element_type=jnp.float32) + bias
        finish_plain(val)

    for d in _DILS:
        @pl.when((kind == _K_RES1) & (dil == d))
        def _res1(d=d):
            build_cat(x_ref, d, True)
            t_ref[PAD:PAD + N_TOK, :] = jnp.dot(
                cat_ref[...], w_ref[0], preferred_element_type=jnp.float32
            ) + bias

    @pl.when(kind == _K_RES2)
    def _res2():
        v = jnp.maximum(t_ref[PAD:PAD + N_TOK, :], 0.0)
        val = jnp.dot(v, w_ref[0, WIDTH:2 * WIDTH, :],
                      preferred_element_type=jnp.float32) + bias
        x_ref[PAD:PAD + N_TOK, :] += val


def _tc_decode(g, big_w, big_b):
    ctrl = jnp.asarray(_CTRL)
    return pl.pallas_call(
        _layer_body,
        grid=(_N_LAYERS,),
        in_specs=[
            pl.BlockSpec(memory_space=pltpu.SMEM),
            pl.BlockSpec((N_TOK, WIDTH), lambda i: (0, 0)),
            pl.BlockSpec((1, K3, WIDTH), lambda i: (i, 0, 0)),
            pl.BlockSpec((_N_LAYERS, WIDTH), lambda i: (0, 0)),
        ],
        out_specs=pl.BlockSpec((N_TOK, WIDTH), lambda i: (0, 0)),
        out_shape=jax.ShapeDtypeStruct((N_TOK, WIDTH), jnp.float32),
        scratch_shapes=[
            pltpu.VMEM((NROW, WIDTH), jnp.float32),
            pltpu.VMEM((NROW, WIDTH), jnp.float32),
            pltpu.VMEM((N_TOK, K3), jnp.float32),
        ],
        compiler_params=pltpu.CompilerParams(
            dimension_semantics=("arbitrary",),
        ),
    )(ctrl, g, big_w, big_b)


def kernel(x, codebook, params):
    idx = x.astype(jnp.int32)
    g = _sc_gather(codebook, idx)
    big_w, big_b = _pack_weights(params)
    out = _tc_decode(g, big_w, big_b)
    return out[:, :INPUT_DIM].reshape(1, N_TOK, INPUT_DIM)


# speedup vs baseline: 2.0733x; 1.6159x over previous
import functools

import jax
import jax.numpy as jnp
import numpy as np
from jax import lax
from jax.experimental import pallas as pl
from jax.experimental.pallas import tpu as pltpu
from jax.experimental.pallas import tpu_sc as plsc

NB_CODE = 512
CODE_DIM = 512
WIDTH = 512
DEPTH = 3
DOWN_T = 3
DRATE = 3
INPUT_DIM = 251
N_TOK = 2048

PAD = 16
NROW = N_TOK + 2 * PAD
K3 = 3 * WIDTH

_NC = 2
_NS = 16
_NW = _NC * _NS
_B_PER_W = N_TOK // _NW

_DILS = tuple(DRATE ** d for d in range(DEPTH))[::-1]

_K_PLAIN_X = 0
_K_PLAIN_OUT = 1
_K_RES1 = 2
_K_RES2 = 3


def _sc_gather(codebook, idx):
    mesh = plsc.VectorSubcoreMesh(core_axis_name="c", subcore_axis_name="s")

    @functools.partial(
        pl.kernel,
        out_type=jax.ShapeDtypeStruct((N_TOK, CODE_DIM), jnp.float32),
        mesh=mesh,
        scratch_types=[
            pltpu.VMEM((_B_PER_W,), jnp.int32),
            pltpu.VMEM((_B_PER_W, CODE_DIM), jnp.float32),
            pltpu.SemaphoreType.DMA,
        ],
    )
    def gather_kernel(table_hbm, idx_hbm, out_hbm, idx_v, rows_v, sem):
        wid = lax.axis_index("s") * _NC + lax.axis_index("c")
        base = wid * _B_PER_W
        pltpu.sync_copy(idx_hbm.at[pl.ds(base, _B_PER_W)], idx_v)
        pltpu.async_copy(table_hbm.at[idx_v], rows_v, sem).wait()
        pltpu.sync_copy(rows_v, out_hbm.at[pl.ds(base, _B_PER_W)])

    return gather_kernel(codebook, idx)


def _layer_schedule():
    layers = [(_K_PLAIN_X, 1, 1)]
    for _ in range(DOWN_T):
        for dil in _DILS:
            layers.append((_K_RES1, dil, 0))
            layers.append((_K_RES2, 1, 0))
        layers.append((_K_PLAIN_X, 1, 0))
    layers.append((_K_PLAIN_X, 1, 1))
    layers.append((_K_PLAIN_OUT, 1, 0))
    return layers


_LAYERS = _layer_schedule()
_N_LAYERS = len(_LAYERS)
_CTRL = np.asarray([[k, d, p] for (k, d, p) in _LAYERS], dtype=np.int32)

_N_K3 = sum(1 for (k, _, _) in _LAYERS if k != _K_RES2)
_N_K1 = _N_LAYERS - _N_K3


def _k3_block_index(i):
    b = (i - 1) // 7
    r = (i - 1) % 7
    return jnp.minimum(1 + 4 * b + (r + 1) // 2, _N_K3 - 1)


def _k1_block_index(i):
    b = (i - 1) // 7
    r = (i - 1) % 7
    return jnp.minimum(3 * b + r // 2, _N_K1 - 1)


def _param_list(params):
    out = [(params['conv_in']['w'], params['conv_in']['b'])]
    for blk in params['blocks']:
        for rb in blk['res']:
            out.append((rb['c1']['w'], rb['c1']['b']))
            out.append((rb['c2']['w'], rb['c2']['b']))
        out.append((blk['conv']['w'], blk['conv']['b']))
    out.append((params['conv_mid']['w'], params['conv_mid']['b']))
    out.append((params['conv_out']['w'], params['conv_out']['b']))
    return out


def _pack_weights(params):
    k3_ws, k1_ws, biases = [], [], []
    for (w, b), (kind, _, _) in zip(_param_list(params), _LAYERS):
        o = w.shape[0]
        if o < WIDTH:
            w = jnp.pad(w, ((0, WIDTH - o), (0, 0), (0, 0)))
            b = jnp.pad(b, (0, WIDTH - o))
        if kind == _K_RES2:
            k1_ws.append(w[:, :, 0])
        else:
            k3_ws.append(w)
        biases.append(b)
    big3 = jnp.stack(k3_ws).transpose(0, 3, 2, 1).reshape(_N_K3, K3, WIDTH)
    big1 = jnp.stack(k1_ws).transpose(0, 2, 1)
    return big3, big1, jnp.stack(biases)


def _layer_body(ctrl_ref, g_ref, w3_ref, w1_ref, b_ref, out_ref,
                x_ref, t_ref, cat_ref):
    i = pl.program_id(0)
    kind = ctrl_ref[i, 0]
    dil = ctrl_ref[i, 1]
    post = ctrl_ref[i, 2]

    @pl.when(i == 0)
    def _init():
        x_ref[...] = jnp.zeros((NROW, WIDTH), jnp.float32)
        t_ref[...] = jnp.zeros((NROW, WIDTH), jnp.float32)
        x_ref[PAD:PAD + N_TOK, :] = g_ref[...]

    bias = b_ref[pl.ds(i, 1), :]

    def finish_plain(val):
        val = jnp.where(post == 1, jnp.maximum(val, 0.0), val)

        @pl.when(kind == _K_PLAIN_X)
        def _():
            x_ref[PAD:PAD + N_TOK, :] = val

        @pl.when(kind == _K_PLAIN_OUT)
        def _():
            out_ref[...] = val

    def build_cat(src_ref, d, with_relu):
        for j, s in enumerate((-d, 0, d)):
            v = src_ref[PAD + s:PAD + s + N_TOK, :]
            if with_relu:
                v = jnp.maximum(v, 0.0)
            cat_ref[:, j * WIDTH:(j + 1) * WIDTH] = v

    @pl.when(kind <= _K_PLAIN_OUT)
    def _plain():
        build_cat(x_ref, 1, False)
        val = jnp.dot(cat_ref[...], w3_ref[0],
                      preferred_element_type=jnp.float32) + bias
        finish_plain(val)

    for d in _DILS:
        @pl.when((kind == _K_RES1) & (dil == d))
        def _res1(d=d):
            build_cat(x_ref, d, True)
            t_ref[PAD:PAD + N_TOK, :] = jnp.dot(
                cat_ref[...], w3_ref[0], preferred_element_type=jnp.float32
            ) + bias

    @pl.when(kind == _K_RES2)
    def _res2():
        v = jnp.maximum(t_ref[PAD:PAD + N_TOK, :], 0.0)
        val = jnp.dot(v, w1_ref[0],
                      preferred_element_type=jnp.float32) + bias
        x_ref[PAD:PAD + N_TOK, :] += val


def _tc_decode(g, big3, big1, big_b):
    ctrl = jnp.asarray(_CTRL)
    return pl.pallas_call(
        _layer_body,
        grid=(_N_LAYERS,),
        in_specs=[
            pl.BlockSpec(memory_space=pltpu.SMEM),
            pl.BlockSpec((N_TOK, WIDTH), lambda i: (0, 0)),
            pl.BlockSpec((1, K3, WIDTH), lambda i: (_k3_block_index(i), 0, 0)),
            pl.BlockSpec((1, WIDTH, WIDTH), lambda i: (_k1_block_index(i), 0, 0)),
            pl.BlockSpec((_N_LAYERS, WIDTH), lambda i: (0, 0)),
        ],
        out_specs=pl.BlockSpec((N_TOK, WIDTH), lambda i: (0, 0)),
        out_shape=jax.ShapeDtypeStruct((N_TOK, WIDTH), jnp.float32),
        scratch_shapes=[
            pltpu.VMEM((NROW, WIDTH), jnp.float32),
            pltpu.VMEM((NROW, WIDTH), jnp.float32),
            pltpu.VMEM((N_TOK, K3), jnp.float32),
        ],
        compiler_params=pltpu.CompilerParams(
            dimension_semantics=("arbitrary",),
        ),
    )(ctrl, g, big3, big1, big_b)


def kernel(x, codebook, params):
    idx = x.astype(jnp.int32)
    g = _sc_gather(codebook, idx)
    big3, big1, big_b = _pack_weights(params)
    out = _tc_decode(g, big3, big1, big_b)
    return out[:, :INPUT_DIM].reshape(1, N_TOK, INPUT_DIM)
